# baseline (speedup 1.0000x reference)
import functools

import jax
import jax.numpy as jnp
from jax import lax
from jax.experimental import pallas as pl
from jax.experimental.pallas import tpu as pltpu
from jax.experimental.pallas import tpu_sc as plsc

NC, NS = 2, 16
NW = NC * NS
CHUNK = 128


@functools.lru_cache(maxsize=None)
def _build(rows, dim, seq_len, rpw, nch):
    mesh = plsc.VectorSubcoreMesh(
        core_axis_name="c", subcore_axis_name="s",
        num_cores=NC, num_subcores=NS)

    @functools.partial(
        pl.kernel,
        out_type=jax.ShapeDtypeStruct((rows, dim), jnp.float32),
        mesh=mesh,
        scratch_types=[
            pltpu.VMEM((nch, CHUNK), jnp.int32),
            pltpu.VMEM((CHUNK, dim), jnp.float32),
            pltpu.SemaphoreType.DMA,
        ],
    )
    def emb(ids_hbm, table_hbm, pos_hbm, out_hbm, idx_v, buf, sem):
        wid = lax.axis_index("s") * NC + lax.axis_index("c")
        base = wid * rpw
        pos_base = base % seq_len
        pltpu.sync_copy(ids_hbm.at[wid], idx_v)
        for c in range(nch):
            pltpu.sync_copy(pos_hbm.at[pl.ds(pos_base + c * CHUNK, CHUNK)], buf)
            pltpu.async_copy(table_hbm.at[idx_v.at[c]], buf, sem,
                             add=True).wait()
            pltpu.sync_copy(buf, out_hbm.at[pl.ds(base + c * CHUNK, CHUNK)])

    return emb


def kernel(input_ids, embedding_weight, pos_embedding):
    batch, seq_len = input_ids.shape
    _, dim = embedding_weight.shape
    rows = batch * seq_len
    rpw = rows // NW
    nch = rpw // CHUNK
    ids = input_ids.reshape(NW, nch, CHUNK).astype(jnp.int32)
    out = _build(rows, dim, seq_len, rpw, nch)(
        ids, embedding_weight, pos_embedding)
    return out.reshape(batch, seq_len, dim)


# speedup vs baseline: 1.2087x; 1.2087x over previous
import functools

import jax
import jax.numpy as jnp
from jax import lax
from jax.experimental import pallas as pl
from jax.experimental.pallas import tpu as pltpu
from jax.experimental.pallas import tpu_sc as plsc

NC, NS = 2, 16
NW = NC * NS
CHUNK = 128


NBUF = 4
POS_AHEAD = 2


@functools.lru_cache(maxsize=None)
def _build(rows, dim, seq_len, rpw, nch):
    mesh = plsc.VectorSubcoreMesh(
        core_axis_name="c", subcore_axis_name="s",
        num_cores=NC, num_subcores=NS)

    @functools.partial(
        pl.kernel,
        out_type=jax.ShapeDtypeStruct((rows, dim), jnp.float32),
        mesh=mesh,
        scratch_types=[
            pltpu.VMEM((nch, CHUNK), jnp.int32),
            pltpu.VMEM((NBUF, CHUNK, dim), jnp.float32),
            pltpu.SemaphoreType.DMA((NBUF,)),
            pltpu.SemaphoreType.DMA((NBUF,)),
            pltpu.SemaphoreType.DMA((NBUF,)),
        ],
    )
    def emb(ids_hbm, table_hbm, pos_hbm, out_hbm, idx_v, bufs,
            pos_sem, gat_sem, out_sem):
        wid = lax.axis_index("s") * NC + lax.axis_index("c")
        base = wid * rpw
        pos_base = base % seq_len
        pltpu.sync_copy(ids_hbm.at[wid], idx_v)

        pend_pos, pend_gat, pend_out = {}, {}, {}

        def start_pos(c):
            b = c % NBUF
            if c - NBUF in pend_out:
                pend_out.pop(c - NBUF).wait()
            pend_pos[c] = pltpu.async_copy(
                pos_hbm.at[pl.ds(pos_base + c * CHUNK, CHUNK)],
                bufs.at[b], pos_sem.at[b])

        for c in range(POS_AHEAD):
            start_pos(c)
        for c in range(nch):
            b = c % NBUF
            if c + POS_AHEAD < nch:
                start_pos(c + POS_AHEAD)
            pend_pos.pop(c).wait()
            pend_gat[c] = pltpu.async_copy(
                table_hbm.at[idx_v.at[c]], bufs.at[b], gat_sem.at[b],
                add=True)
            pend_gat.pop(c).wait()
            pend_out[c] = pltpu.async_copy(
                bufs.at[b], out_hbm.at[pl.ds(base + c * CHUNK, CHUNK)],
                out_sem.at[b])
        for c in sorted(pend_out):
            pend_out.pop(c).wait()

    return emb


def kernel(input_ids, embedding_weight, pos_embedding):
    batch, seq_len = input_ids.shape
    _, dim = embedding_weight.shape
    rows = batch * seq_len
    rpw = rows // NW
    nch = rpw // CHUNK
    ids = input_ids.reshape(NW, nch, CHUNK).astype(jnp.int32)
    out = _build(rows, dim, seq_len, rpw, nch)(
        ids, embedding_weight, pos_embedding)
    return out.reshape(batch, seq_len, dim)


# speedup vs baseline: 1.2379x; 1.0242x over previous
import functools

import jax
import jax.numpy as jnp
from jax import lax
from jax.experimental import pallas as pl
from jax.experimental.pallas import tpu as pltpu
from jax.experimental.pallas import tpu_sc as plsc

NC, NS = 2, 16
NW = NC * NS
CHUNK = 128


NBUF = 4
POS_AHEAD = 2
GAT_LAG = 1


@functools.lru_cache(maxsize=None)
def _build(rows, dim, seq_len, rpw, nch):
    mesh = plsc.VectorSubcoreMesh(
        core_axis_name="c", subcore_axis_name="s",
        num_cores=NC, num_subcores=NS)

    @functools.partial(
        pl.kernel,
        out_type=jax.ShapeDtypeStruct((rows, dim), jnp.float32),
        mesh=mesh,
        scratch_types=[
            pltpu.VMEM((nch, CHUNK), jnp.int32),
            pltpu.VMEM((NBUF, CHUNK, dim), jnp.float32),
            pltpu.SemaphoreType.DMA((NBUF,)),
            pltpu.SemaphoreType.DMA((NBUF,)),
            pltpu.SemaphoreType.DMA((NBUF,)),
        ],
    )
    def emb(ids_hbm, table_hbm, pos_hbm, out_hbm, idx_v, bufs,
            pos_sem, gat_sem, out_sem):
        wid = lax.axis_index("s") * NC + lax.axis_index("c")
        base = wid * rpw
        pos_base = base % seq_len
        pltpu.sync_copy(ids_hbm.at[wid], idx_v)

        pend_pos, pend_gat, pend_out = {}, {}, {}

        def start_pos(c):
            b = c % NBUF
            if c - NBUF in pend_out:
                pend_out.pop(c - NBUF).wait()
            pend_pos[c] = pltpu.async_copy(
                pos_hbm.at[pl.ds(pos_base + c * CHUNK, CHUNK)],
                bufs.at[b], pos_sem.at[b])

        def start_gather(c):
            b = c % NBUF
            pend_pos.pop(c).wait()
            pend_gat[c] = pltpu.async_copy(
                table_hbm.at[idx_v.at[c]], bufs.at[b], gat_sem.at[b],
                add=True)

        def start_out(c):
            b = c % NBUF
            pend_gat.pop(c).wait()
            pend_out[c] = pltpu.async_copy(
                bufs.at[b], out_hbm.at[pl.ds(base + c * CHUNK, CHUNK)],
                out_sem.at[b])

        for c in range(POS_AHEAD):
            start_pos(c)
        for c in range(nch + GAT_LAG):
            if c + POS_AHEAD < nch:
                start_pos(c + POS_AHEAD)
            if c < nch:
                start_gather(c)
            if c - GAT_LAG >= 0:
                start_out(c - GAT_LAG)
        for c in sorted(pend_out):
            pend_out.pop(c).wait()

    return emb


def kernel(input_ids, embedding_weight, pos_embedding):
    batch, seq_len = input_ids.shape
    _, dim = embedding_weight.shape
    rows = batch * seq_len
    rpw = rows // NW
    nch = rpw // CHUNK
    ids = input_ids.reshape(NW, nch, CHUNK).astype(jnp.int32)
    out = _build(rows, dim, seq_len, rpw, nch)(
        ids, embedding_weight, pos_embedding)
    return out.reshape(batch, seq_len, dim)


# speedup vs baseline: 1.2954x; 1.0464x over previous
import functools

import jax
import jax.numpy as jnp
from jax import lax
from jax.experimental import pallas as pl
from jax.experimental.pallas import tpu as pltpu
from jax.experimental.pallas import tpu_sc as plsc

NC, NS = 2, 16
NW = NC * NS
CHUNK = 128
NBUF = 4
LOC_AHEAD = 2
GAT_LAG = 1


@functools.lru_cache(maxsize=None)
def _build(batch, seq_len, dim):
    rows = batch * seq_len
    span = seq_len // NW
    nch = (batch * span) // CHUNK
    ch_per_b = span // CHUNK
    mesh = plsc.VectorSubcoreMesh(
        core_axis_name="c", subcore_axis_name="s",
        num_cores=NC, num_subcores=NS)

    @functools.partial(
        pl.kernel,
        out_type=jax.ShapeDtypeStruct((rows, dim), jnp.float32),
        mesh=mesh,
        scratch_types=[
            pltpu.VMEM((nch, CHUNK), jnp.int32),
            pltpu.VMEM_SHARED((NS, span, dim), jnp.float32),
            pltpu.VMEM((NBUF, CHUNK, dim), jnp.float32),
            pltpu.SemaphoreType.DMA,
            pltpu.SemaphoreType.DMA((NBUF,)),
            pltpu.SemaphoreType.DMA((NBUF,)),
            pltpu.SemaphoreType.DMA((NBUF,)),
        ],
    )
    def emb(ids_hbm, table_hbm, pos_hbm, out_hbm, idx_v, pos_shr, bufs,
            ld_sem, loc_sem, gat_sem, out_sem):
        sid = lax.axis_index("s")
        wid = sid * NC + lax.axis_index("c")
        s0 = wid * span

        pos_ld = pltpu.async_copy(pos_hbm.at[pl.ds(s0, span)],
                                  pos_shr.at[sid], ld_sem)
        for c in range(nch):
            b_row, half = divmod(c, ch_per_b)
            pltpu.sync_copy(
                ids_hbm.at[b_row, pl.ds(s0 + half * CHUNK, CHUNK)],
                idx_v.at[c])
        pos_ld.wait()

        pend_loc, pend_gat, pend_out = {}, {}, {}

        def flat_base(c):
            b_row, half = divmod(c, ch_per_b)
            return b_row * seq_len + s0 + half * CHUNK

        def start_loc(c):
            b = c % NBUF
            if c - NBUF in pend_out:
                pend_out.pop(c - NBUF).wait()
            half = c % ch_per_b
            pend_loc[c] = pltpu.async_copy(
                pos_shr.at[sid, pl.ds(half * CHUNK, CHUNK)],
                bufs.at[b], loc_sem.at[b])

        def start_gather(c):
            b = c % NBUF
            pend_loc.pop(c).wait()
            pend_gat[c] = pltpu.async_copy(
                table_hbm.at[idx_v.at[c]], bufs.at[b], gat_sem.at[b],
                add=True)

        def start_out(c):
            b = c % NBUF
            pend_gat.pop(c).wait()
            pend_out[c] = pltpu.async_copy(
                bufs.at[b], out_hbm.at[pl.ds(flat_base(c), CHUNK)],
                out_sem.at[b])

        for c in range(LOC_AHEAD):
            start_loc(c)
        for c in range(nch + GAT_LAG):
            if c + LOC_AHEAD < nch:
                start_loc(c + LOC_AHEAD)
            if c < nch:
                start_gather(c)
            if c - GAT_LAG >= 0:
                start_out(c - GAT_LAG)
        for c in sorted(pend_out):
            pend_out.pop(c).wait()

    return emb


def kernel(input_ids, embedding_weight, pos_embedding):
    batch, seq_len = input_ids.shape
    _, dim = embedding_weight.shape
    ids = input_ids.astype(jnp.int32)
    out = _build(batch, seq_len, dim)(ids, embedding_weight, pos_embedding)
    return out.reshape(batch, seq_len, dim)


# speedup vs baseline: 1.4280x; 1.1024x over previous
import functools

import jax
import jax.numpy as jnp
from jax import lax
from jax.experimental import pallas as pl
from jax.experimental.pallas import tpu as pltpu
from jax.experimental.pallas import tpu_sc as plsc

NC, NS = 2, 16
NW = NC * NS
LANES = 16
CHUNK = 128
NBUF = 4
GAT_AHEAD = 2


@functools.lru_cache(maxsize=None)
def _build(batch, seq_len, dim):
    rows = batch * seq_len
    span = seq_len // NW
    nch = (batch * span) // CHUNK
    ch_per_b = span // CHUNK
    nvec = dim // LANES
    mesh = plsc.VectorSubcoreMesh(
        core_axis_name="c", subcore_axis_name="s",
        num_cores=NC, num_subcores=NS)

    @functools.partial(
        pl.kernel,
        out_type=jax.ShapeDtypeStruct((rows, dim), jnp.float32),
        mesh=mesh,
        scratch_types=[
            pltpu.VMEM((nch, CHUNK), jnp.int32),
            pltpu.VMEM((span, dim), jnp.float32),
            pltpu.VMEM((NBUF, CHUNK, dim), jnp.float32),
            pltpu.SemaphoreType.DMA,
            pltpu.SemaphoreType.DMA,
            pltpu.SemaphoreType.DMA((NBUF,)),
            pltpu.SemaphoreType.DMA((NBUF,)),
        ],
    )
    def emb(ids_hbm, table_hbm, pos_hbm, out_hbm, idx_v, pos_v, bufs,
            id_sem, ld_sem, gat_sem, out_sem):
        wid = lax.axis_index("s") * NC + lax.axis_index("c")
        s0 = wid * span

        pos_ld = pltpu.async_copy(pos_hbm.at[pl.ds(s0, span)], pos_v, ld_sem)
        id_lds = []
        for c in range(nch):
            b_row, half = divmod(c, ch_per_b)
            id_lds.append(pltpu.async_copy(
                ids_hbm.at[b_row, pl.ds(s0 + half * CHUNK, CHUNK)],
                idx_v.at[c], id_sem))
        for d in id_lds:
            d.wait()

        pend_gat, pend_out = {}, {}

        def flat_base(c):
            b_row, half = divmod(c, ch_per_b)
            return b_row * seq_len + s0 + half * CHUNK

        def start_gather(c):
            b = c % NBUF
            if c - NBUF in pend_out:
                pend_out.pop(c - NBUF).wait()
            pend_gat[c] = pltpu.async_copy(
                table_hbm.at[idx_v.at[c]], bufs.at[b], gat_sem.at[b])

        def add_pos(c):
            b = c % NBUF
            half = c % ch_per_b
            buf = bufs.at[b]

            def body(r, carry):
                pr = half * CHUNK + r
                for j in range(nvec):
                    sl = pl.ds(j * LANES, LANES)
                    plsc.addupdate(buf.at[r, sl], pos_v[pr, sl])
                return carry

            lax.fori_loop(0, CHUNK, body, 0)

        def finish_chunk(c):
            b = c % NBUF
            pend_gat.pop(c).wait()
            add_pos(c)
            pend_out[c] = pltpu.async_copy(
                bufs.at[b], out_hbm.at[pl.ds(flat_base(c), CHUNK)],
                out_sem.at[b])

        for c in range(GAT_AHEAD):
            start_gather(c)
        pos_ld.wait()
        for c in range(nch):
            if c + GAT_AHEAD < nch:
                start_gather(c + GAT_AHEAD)
            finish_chunk(c)
        for c in sorted(pend_out):
            pend_out.pop(c).wait()

    return emb


def kernel(input_ids, embedding_weight, pos_embedding):
    batch, seq_len = input_ids.shape
    _, dim = embedding_weight.shape
    ids = input_ids.astype(jnp.int32)
    out = _build(batch, seq_len, dim)(ids, embedding_weight, pos_embedding)
    return out.reshape(batch, seq_len, dim)
